# baseline (speedup 1.0000x reference)
import jax
import jax.numpy as jnp
from jax.experimental import pallas as pl
from jax.experimental.pallas import tpu as pltpu

_DIM = 64


def _zero_fill(out_ref):
    out_ref[...] = jnp.zeros_like(out_ref)


def kernel(tgt, times):
    del times
    batch = tgt.shape[0]
    rows = batch * _DIM // 128
    grid = 8
    out = pl.pallas_call(
        _zero_fill,
        out_shape=jax.ShapeDtypeStruct((rows, 128), jnp.float32),
        grid=(grid,),
        out_specs=pl.BlockSpec((rows // grid, 128), lambda i: (i, 0)),
        compiler_params=pltpu.CompilerParams(
            dimension_semantics=("parallel",)),
    )()
    return out.reshape(batch, _DIM)


# speedup vs baseline: 1.5662x; 1.5662x over previous
import jax
import jax.numpy as jnp
from jax.experimental import pallas as pl
from jax.experimental.pallas import tpu as pltpu

_DIM = 64


def _zero_fill(out_ref):
    out_ref[...] = jnp.zeros_like(out_ref)


def kernel(tgt, times):
    del times
    batch = tgt.shape[0]
    grid = 8
    return pl.pallas_call(
        _zero_fill,
        out_shape=jax.ShapeDtypeStruct((batch, _DIM), jnp.float32),
        grid=(grid,),
        out_specs=pl.BlockSpec((batch // grid, _DIM), lambda i: (i, 0)),
        compiler_params=pltpu.CompilerParams(
            dimension_semantics=("parallel",)),
    )()
